# baseline (speedup 1.0000x reference)
import functools

import jax
import jax.numpy as jnp
from jax import lax
from jax.experimental import pallas as pl
from jax.experimental.pallas import tpu as pltpu
from jax.experimental.pallas import tpu_sc as plsc

NUM_USERS = 5000
NUM_ITEMS = 5000
D = 128
DH = D // 2
L_LAYERS = 3
E = 320000
N = NUM_USERS + NUM_ITEMS
N_PAD = 10240
DUMMY = N

NC = 2
NS = 16
NW = NC * NS
B = 128
NB = 80
NBUF = 4
E_PAD = NW * B * NB
ROWS_PER_TILE = N_PAD // NS


def _edge_body(xlo_hbm, xhi_hbm, s_hbm, t_hbm, src_hbm, dst_hbm, zeros_hbm,
               out_hbm,
               s_tab, t_tab, src_all, dst_all, score_all,
               r0, r1, r2, r3, agg,
               ga0, ga1, ga2, ga3, sa0, sa1, sa2, sa3):
    rowbufs = (r0, r1, r2, r3)
    gsems = (ga0, ga1, ga2, ga3)
    ssems = (sa0, sa1, sa2, sa3)
    core = lax.axis_index("c")
    sid = lax.axis_index("s")
    wid = sid * NC + core

    pltpu.sync_copy(s_hbm, s_tab)
    pltpu.sync_copy(t_hbm, t_tab)
    pltpu.sync_copy(src_hbm.at[pl.ds(wid * NB, NB)], src_all)
    pltpu.sync_copy(dst_hbm.at[pl.ds(wid * NB, NB)], dst_all)

    def score_batch(b, carry):
        for i in range(B // 16):
            si = src_all[b, pl.ds(i * 16, 16)]
            di = dst_all[b, pl.ds(i * 16, 16)]
            z = plsc.load_gather(s_tab, [si]) + plsc.load_gather(t_tab, [di])
            score_all[b, pl.ds(i * 16, 16)] = 1.0 / (1.0 + jnp.exp(-z))
        return carry

    lax.fori_loop(0, NB, score_batch, 0)

    def scale_rows(rows, b):
        def body(g, carry):
            sv = score_all[b, pl.ds(g * 16, 16)]
            for k in range(16):
                e = g * 16 + k
                sc = sv[k]
                for j in range(DH // 16):
                    sl = pl.ds(j * 16, 16)
                    rows[e, sl] = rows[e, sl] * sc
            return carry
        lax.fori_loop(0, B // 16, body, 0)

    rtile = pl.ds(sid * ROWS_PER_TILE, ROWS_PER_TILE)

    for half, x_hbm in enumerate((xlo_hbm, xhi_hbm)):
        def issue_gather(b, rows, gsem):
            pltpu.async_copy(x_hbm.at[src_all.at[b]], rows, gsem)

        def wait_gather(rows, gsem):
            pltpu.make_async_copy(x_hbm.at[src_all.at[0]], rows, gsem).wait()

        def issue_scatter(b, rows, ssem):
            pltpu.async_copy(rows, agg.at[dst_all.at[b]], ssem, add=True)

        def wait_scatter(rows, ssem):
            pltpu.make_async_copy(rows, agg.at[dst_all.at[0]], ssem).wait()

        pltpu.sync_copy(zeros_hbm, agg.at[rtile])
        plsc.subcore_barrier()

        bufs = list(zip(rowbufs, gsems, ssems))

        for k in range(3):
            issue_gather(k, bufs[k][0], bufs[k][1])

        for k in range(NBUF):
            rows, gsem, ssem = bufs[k]
            wait_gather(rows, gsem)
            scale_rows(rows, k)
            issue_scatter(k, rows, ssem)
            r = (k + 3) % NBUF
            rrows, rgsem, rssem = bufs[r]
            if k >= NBUF - 3:
                wait_scatter(rrows, rssem)
            issue_gather(k + 3, rrows, rgsem)

        def round_body(t, carry):
            for k in range(NBUF):
                m = NBUF * t + k
                rows, gsem, ssem = bufs[k]
                wait_gather(rows, gsem)
                scale_rows(rows, m)
                issue_scatter(m, rows, ssem)
                r = (k + 3) % NBUF
                rrows, rgsem, rssem = bufs[r]
                wait_scatter(rrows, rssem)
                issue_gather(m + 3, rrows, rgsem)
            return carry

        lax.fori_loop(1, NB // NBUF - 1, round_body, 0)

        for k in range(NBUF):
            m = NB - NBUF + k
            rows, gsem, ssem = bufs[k]
            wait_gather(rows, gsem)
            scale_rows(rows, m)
            issue_scatter(m, rows, ssem)
            r = (k + 3) % NBUF
            rrows, rgsem, rssem = bufs[r]
            wait_scatter(rrows, rssem)
            if k < NBUF - 3:
                issue_gather(m + 3, rrows, rgsem)

        last = bufs[(NB - 1) % NBUF]
        wait_scatter(last[0], last[2])
        plsc.subcore_barrier()

        pltpu.sync_copy(agg.at[rtile], out_hbm.at[core, half, rtile])
        plsc.subcore_barrier()


@functools.cache
def _edge_kernel_fn():
    return functools.partial(
        pl.kernel,
        out_type=jax.ShapeDtypeStruct((NC, 2, N_PAD, DH), jnp.float32),
        mesh=plsc.VectorSubcoreMesh(core_axis_name="c", subcore_axis_name="s"),
        compiler_params=pltpu.CompilerParams(needs_layout_passes=False,
                                             use_tc_tiling_on_sc=False),
        scratch_types=[
            pltpu.VMEM((N_PAD,), jnp.float32),
            pltpu.VMEM((N_PAD,), jnp.float32),
            pltpu.VMEM((NB, B), jnp.int32),
            pltpu.VMEM((NB, B), jnp.int32),
            pltpu.VMEM((NB, B), jnp.float32),
        ] + [pltpu.VMEM((B, DH), jnp.float32) for _ in range(NBUF)]
          + [pltpu.VMEM_SHARED((N_PAD, DH), jnp.float32)]
          + [pltpu.SemaphoreType.DMA for _ in range(2 * NBUF)],
    )(_edge_body)


BLK = 1024
GRID = N_PAD // BLK


def _update_body(x_ref, p00_ref, p01_ref, p10_ref, p11_ref, w_ref, b_ref,
                 ws_ref, wt_ref, bias_ref,
                 xo_ref, xlo_ref, xhi_ref, so_ref, to_ref):
    x = x_ref[...]
    agg = jnp.concatenate(
        [p00_ref[0, 0] + p10_ref[0, 0], p01_ref[0, 0] + p11_ref[0, 0]],
        axis=1)
    h = jnp.dot(x, w_ref[0], preferred_element_type=jnp.float32)
    h = h + jnp.dot(agg, w_ref[1], preferred_element_type=jnp.float32)
    h = h + b_ref[...]
    xn = jnp.maximum(h, 0.0)
    xo_ref[...] = xn
    xlo_ref[...] = xn[:, :DH]
    xhi_ref[...] = xn[:, DH:]
    so_ref[...] = jnp.sum(xn * ws_ref[...], axis=1)
    to_ref[...] = jnp.sum(xn * wt_ref[...], axis=1) + bias_ref[0, 0]


def _update(x, parts, w2, bcast, ws, wt, bias):
    return pl.pallas_call(
        _update_body,
        grid=(GRID,),
        in_specs=[
            pl.BlockSpec((BLK, D), lambda i: (i, 0)),
            pl.BlockSpec((1, 1, BLK, DH), lambda i: (0, 0, i, 0)),
            pl.BlockSpec((1, 1, BLK, DH), lambda i: (0, 1, i, 0)),
            pl.BlockSpec((1, 1, BLK, DH), lambda i: (1, 0, i, 0)),
            pl.BlockSpec((1, 1, BLK, DH), lambda i: (1, 1, i, 0)),
            pl.BlockSpec((2, D, D), lambda i: (0, 0, 0)),
            pl.BlockSpec((1, D), lambda i: (0, 0)),
            pl.BlockSpec((1, D), lambda i: (0, 0)),
            pl.BlockSpec((1, D), lambda i: (0, 0)),
            pl.BlockSpec((1, 1), lambda i: (0, 0)),
        ],
        out_specs=[
            pl.BlockSpec((BLK, D), lambda i: (i, 0)),
            pl.BlockSpec((BLK, DH), lambda i: (i, 0)),
            pl.BlockSpec((BLK, DH), lambda i: (i, 0)),
            pl.BlockSpec((BLK,), lambda i: (i,)),
            pl.BlockSpec((BLK,), lambda i: (i,)),
        ],
        out_shape=[
            jax.ShapeDtypeStruct((N_PAD, D), jnp.float32),
            jax.ShapeDtypeStruct((N_PAD, DH), jnp.float32),
            jax.ShapeDtypeStruct((N_PAD, DH), jnp.float32),
            jax.ShapeDtypeStruct((N_PAD,), jnp.float32),
            jax.ShapeDtypeStruct((N_PAD,), jnp.float32),
        ],
    )(x, parts, parts, parts, parts, w2, bcast, ws, wt, bias)


def _proj_body(x_ref, ws_ref, wt_ref, bias_ref, so_ref, to_ref):
    x = x_ref[...]
    so_ref[...] = jnp.sum(x * ws_ref[...], axis=1)
    to_ref[...] = jnp.sum(x * wt_ref[...], axis=1) + bias_ref[0, 0]


def _proj(x, ws, wt, bias):
    return pl.pallas_call(
        _proj_body,
        grid=(GRID,),
        in_specs=[
            pl.BlockSpec((BLK, D), lambda i: (i, 0)),
            pl.BlockSpec((1, D), lambda i: (0, 0)),
            pl.BlockSpec((1, D), lambda i: (0, 0)),
            pl.BlockSpec((1, 1), lambda i: (0, 0)),
        ],
        out_specs=[
            pl.BlockSpec((BLK,), lambda i: (i,)),
            pl.BlockSpec((BLK,), lambda i: (i,)),
        ],
        out_shape=[
            jax.ShapeDtypeStruct((N_PAD,), jnp.float32),
            jax.ShapeDtypeStruct((N_PAD,), jnp.float32),
        ],
    )(x, ws, wt, bias)


def kernel(edge_index, user_emb, item_emb, att_w, att_b, agg_w, agg_b):
    src = edge_index[0]
    dst = edge_index[1]
    pad = E_PAD - E
    src_p = jnp.concatenate([src, jnp.zeros((pad,), jnp.int32)]).reshape(NW * NB, B)
    dst_p = jnp.concatenate([dst, jnp.full((pad,), DUMMY, jnp.int32)]).reshape(NW * NB, B)
    zeros = jnp.zeros((ROWS_PER_TILE, DH), jnp.float32)

    x = jnp.concatenate(
        [user_emb, item_emb, jnp.zeros((N_PAD - N, D), jnp.float32)], axis=0)
    xlo = x[:, :DH]
    xhi = x[:, DH:]

    ws = [att_w[l, :D, 0].reshape(1, D) for l in range(L_LAYERS)]
    wt = [att_w[l, D:, 0].reshape(1, D) for l in range(L_LAYERS)]
    bs = [att_b[l].reshape(1, 1) for l in range(L_LAYERS)]
    w2 = [agg_w[l].reshape(2, D, D) for l in range(L_LAYERS)]
    bc = [agg_b[l].reshape(1, D) for l in range(L_LAYERS)]

    s, t = _proj(x, ws[0], wt[0], bs[0])
    for l in range(L_LAYERS):
        parts = _edge_kernel_fn()(xlo, xhi, s, t, src_p, dst_p, zeros)
        nl = min(l + 1, L_LAYERS - 1)
        x, xlo, xhi, s, t = _update(x, parts, w2[l], bc[l], ws[nl], wt[nl], bs[nl])

    return (x[:NUM_USERS], x[NUM_USERS:N])


# speedup vs baseline: 1.0241x; 1.0241x over previous
import functools

import jax
import jax.numpy as jnp
from jax import lax
from jax.experimental import pallas as pl
from jax.experimental.pallas import tpu as pltpu
from jax.experimental.pallas import tpu_sc as plsc

NUM_USERS = 5000
NUM_ITEMS = 5000
D = 128
DH = D // 2
L_LAYERS = 3
E = 320000
N = NUM_USERS + NUM_ITEMS
N_PAD = 10240
DUMMY = N

NC = 2
NS = 16
NW = NC * NS
B = 128
NB = 80
NBUF = 5
LA = 4
E_PAD = NW * B * NB
ROWS_PER_TILE = N_PAD // NS


def _edge_body(xlo_hbm, xhi_hbm, s_hbm, t_hbm, src_hbm, dst_hbm, zeros_hbm,
               out_hbm,
               s_tab, t_tab, src_all, dst_all,
               r0, r1, r2, r3, r4, agg,
               ga0, ga1, ga2, ga3, ga4, sa0, sa1, sa2, sa3, sa4):
    rowbufs = (r0, r1, r2, r3, r4)
    gsems = (ga0, ga1, ga2, ga3, ga4)
    ssems = (sa0, sa1, sa2, sa3, sa4)
    core = lax.axis_index("c")
    sid = lax.axis_index("s")
    wid = sid * NC + core

    pltpu.sync_copy(s_hbm, s_tab)
    pltpu.sync_copy(t_hbm, t_tab)
    pltpu.sync_copy(src_hbm.at[pl.ds(wid * NB, NB)], src_all)
    pltpu.sync_copy(dst_hbm.at[pl.ds(wid * NB, NB)], dst_all)

    def scale_rows(rows, b):
        def body(g, carry):
            si = src_all[b, pl.ds(g * 16, 16)]
            di = dst_all[b, pl.ds(g * 16, 16)]
            z = plsc.load_gather(s_tab, [si]) + plsc.load_gather(t_tab, [di])
            sv = 1.0 / (1.0 + jnp.exp(-z))
            for k in range(16):
                e = g * 16 + k
                sc = sv[k]
                for j in range(DH // 16):
                    sl = pl.ds(j * 16, 16)
                    rows[e, sl] = rows[e, sl] * sc
            return carry
        lax.fori_loop(0, B // 16, body, 0)

    rtile = pl.ds(sid * ROWS_PER_TILE, ROWS_PER_TILE)

    for half, x_hbm in enumerate((xlo_hbm, xhi_hbm)):
        def issue_gather(b, rows, gsem):
            pltpu.async_copy(x_hbm.at[src_all.at[b]], rows, gsem)

        def wait_gather(rows, gsem):
            pltpu.make_async_copy(x_hbm.at[src_all.at[0]], rows, gsem).wait()

        def issue_scatter(b, rows, ssem):
            pltpu.async_copy(rows, agg.at[dst_all.at[b]], ssem, add=True)

        def wait_scatter(rows, ssem):
            pltpu.make_async_copy(rows, agg.at[dst_all.at[0]], ssem).wait()

        pltpu.sync_copy(zeros_hbm, agg.at[rtile])
        plsc.subcore_barrier()

        bufs = list(zip(rowbufs, gsems, ssems))

        for k in range(LA):
            issue_gather(k, bufs[k][0], bufs[k][1])

        for k in range(NBUF):
            rows, gsem, ssem = bufs[k]
            wait_gather(rows, gsem)
            scale_rows(rows, k)
            issue_scatter(k, rows, ssem)
            r = (k + LA) % NBUF
            rrows, rgsem, rssem = bufs[r]
            if k >= NBUF - LA:
                wait_scatter(rrows, rssem)
            issue_gather(k + LA, rrows, rgsem)

        def round_body(t, carry):
            for k in range(NBUF):
                m = NBUF * t + k
                rows, gsem, ssem = bufs[k]
                wait_gather(rows, gsem)
                scale_rows(rows, m)
                issue_scatter(m, rows, ssem)
                r = (k + LA) % NBUF
                rrows, rgsem, rssem = bufs[r]
                wait_scatter(rrows, rssem)
                issue_gather(m + LA, rrows, rgsem)
            return carry

        lax.fori_loop(1, NB // NBUF - 1, round_body, 0)

        for k in range(NBUF):
            m = NB - NBUF + k
            rows, gsem, ssem = bufs[k]
            wait_gather(rows, gsem)
            scale_rows(rows, m)
            issue_scatter(m, rows, ssem)
            r = (k + LA) % NBUF
            rrows, rgsem, rssem = bufs[r]
            wait_scatter(rrows, rssem)
            if k < NBUF - LA:
                issue_gather(m + LA, rrows, rgsem)

        last = bufs[(NB - 1) % NBUF]
        wait_scatter(last[0], last[2])
        plsc.subcore_barrier()

        pltpu.sync_copy(agg.at[rtile], out_hbm.at[core, half, rtile])
        plsc.subcore_barrier()


@functools.cache
def _edge_kernel_fn():
    return functools.partial(
        pl.kernel,
        out_type=jax.ShapeDtypeStruct((NC, 2, N_PAD, DH), jnp.float32),
        mesh=plsc.VectorSubcoreMesh(core_axis_name="c", subcore_axis_name="s"),
        compiler_params=pltpu.CompilerParams(needs_layout_passes=False,
                                             use_tc_tiling_on_sc=False),
        scratch_types=[
            pltpu.VMEM((N_PAD,), jnp.float32),
            pltpu.VMEM((N_PAD,), jnp.float32),
            pltpu.VMEM((NB, B), jnp.int32),
            pltpu.VMEM((NB, B), jnp.int32),
        ] + [pltpu.VMEM((B, DH), jnp.float32) for _ in range(NBUF)]
          + [pltpu.VMEM_SHARED((N_PAD, DH), jnp.float32)]
          + [pltpu.SemaphoreType.DMA for _ in range(2 * NBUF)],
    )(_edge_body)


BLK = 1024
GRID = N_PAD // BLK


def _update_body(x_ref, p00_ref, p01_ref, p10_ref, p11_ref, w_ref, b_ref,
                 ws_ref, wt_ref, bias_ref,
                 xo_ref, xlo_ref, xhi_ref, so_ref, to_ref):
    x = x_ref[...]
    agg = jnp.concatenate(
        [p00_ref[0, 0] + p10_ref[0, 0], p01_ref[0, 0] + p11_ref[0, 0]],
        axis=1)
    h = jnp.dot(x, w_ref[0], preferred_element_type=jnp.float32)
    h = h + jnp.dot(agg, w_ref[1], preferred_element_type=jnp.float32)
    h = h + b_ref[...]
    xn = jnp.maximum(h, 0.0)
    xo_ref[...] = xn
    xlo_ref[...] = xn[:, :DH]
    xhi_ref[...] = xn[:, DH:]
    so_ref[...] = jnp.sum(xn * ws_ref[...], axis=1)
    to_ref[...] = jnp.sum(xn * wt_ref[...], axis=1) + bias_ref[0, 0]


def _update(x, parts, w2, bcast, ws, wt, bias):
    return pl.pallas_call(
        _update_body,
        grid=(GRID,),
        in_specs=[
            pl.BlockSpec((BLK, D), lambda i: (i, 0)),
            pl.BlockSpec((1, 1, BLK, DH), lambda i: (0, 0, i, 0)),
            pl.BlockSpec((1, 1, BLK, DH), lambda i: (0, 1, i, 0)),
            pl.BlockSpec((1, 1, BLK, DH), lambda i: (1, 0, i, 0)),
            pl.BlockSpec((1, 1, BLK, DH), lambda i: (1, 1, i, 0)),
            pl.BlockSpec((2, D, D), lambda i: (0, 0, 0)),
            pl.BlockSpec((1, D), lambda i: (0, 0)),
            pl.BlockSpec((1, D), lambda i: (0, 0)),
            pl.BlockSpec((1, D), lambda i: (0, 0)),
            pl.BlockSpec((1, 1), lambda i: (0, 0)),
        ],
        out_specs=[
            pl.BlockSpec((BLK, D), lambda i: (i, 0)),
            pl.BlockSpec((BLK, DH), lambda i: (i, 0)),
            pl.BlockSpec((BLK, DH), lambda i: (i, 0)),
            pl.BlockSpec((BLK,), lambda i: (i,)),
            pl.BlockSpec((BLK,), lambda i: (i,)),
        ],
        out_shape=[
            jax.ShapeDtypeStruct((N_PAD, D), jnp.float32),
            jax.ShapeDtypeStruct((N_PAD, DH), jnp.float32),
            jax.ShapeDtypeStruct((N_PAD, DH), jnp.float32),
            jax.ShapeDtypeStruct((N_PAD,), jnp.float32),
            jax.ShapeDtypeStruct((N_PAD,), jnp.float32),
        ],
    )(x, parts, parts, parts, parts, w2, bcast, ws, wt, bias)


def _proj_body(x_ref, ws_ref, wt_ref, bias_ref, so_ref, to_ref):
    x = x_ref[...]
    so_ref[...] = jnp.sum(x * ws_ref[...], axis=1)
    to_ref[...] = jnp.sum(x * wt_ref[...], axis=1) + bias_ref[0, 0]


def _proj(x, ws, wt, bias):
    return pl.pallas_call(
        _proj_body,
        grid=(GRID,),
        in_specs=[
            pl.BlockSpec((BLK, D), lambda i: (i, 0)),
            pl.BlockSpec((1, D), lambda i: (0, 0)),
            pl.BlockSpec((1, D), lambda i: (0, 0)),
            pl.BlockSpec((1, 1), lambda i: (0, 0)),
        ],
        out_specs=[
            pl.BlockSpec((BLK,), lambda i: (i,)),
            pl.BlockSpec((BLK,), lambda i: (i,)),
        ],
        out_shape=[
            jax.ShapeDtypeStruct((N_PAD,), jnp.float32),
            jax.ShapeDtypeStruct((N_PAD,), jnp.float32),
        ],
    )(x, ws, wt, bias)


def kernel(edge_index, user_emb, item_emb, att_w, att_b, agg_w, agg_b):
    src = edge_index[0]
    dst = edge_index[1]
    pad = E_PAD - E
    src_p = jnp.concatenate([src, jnp.zeros((pad,), jnp.int32)]).reshape(NW * NB, B)
    dst_p = jnp.concatenate([dst, jnp.full((pad,), DUMMY, jnp.int32)]).reshape(NW * NB, B)
    zeros = jnp.zeros((ROWS_PER_TILE, DH), jnp.float32)

    x = jnp.concatenate(
        [user_emb, item_emb, jnp.zeros((N_PAD - N, D), jnp.float32)], axis=0)
    xlo = x[:, :DH]
    xhi = x[:, DH:]

    ws = [att_w[l, :D, 0].reshape(1, D) for l in range(L_LAYERS)]
    wt = [att_w[l, D:, 0].reshape(1, D) for l in range(L_LAYERS)]
    bs = [att_b[l].reshape(1, 1) for l in range(L_LAYERS)]
    w2 = [agg_w[l].reshape(2, D, D) for l in range(L_LAYERS)]
    bc = [agg_b[l].reshape(1, D) for l in range(L_LAYERS)]

    s, t = _proj(x, ws[0], wt[0], bs[0])
    for l in range(L_LAYERS):
        parts = _edge_kernel_fn()(xlo, xhi, s, t, src_p, dst_p, zeros)
        nl = min(l + 1, L_LAYERS - 1)
        x, xlo, xhi, s, t = _update(x, parts, w2[l], bc[l], ws[nl], wt[nl], bs[nl])

    return (x[:NUM_USERS], x[NUM_USERS:N])


# speedup vs baseline: 1.4986x; 1.4633x over previous
import functools

import jax
import jax.numpy as jnp
from jax import lax
from jax.experimental import pallas as pl
from jax.experimental.pallas import tpu as pltpu
from jax.experimental.pallas import tpu_sc as plsc

NUM_USERS = 5000
NUM_ITEMS = 5000
D = 128
DH = D // 2
L_LAYERS = 3
E = 320000
N = NUM_USERS + NUM_ITEMS
N_PAD = 10240
N_TAB = 10048
DUMMY = N

NC = 2
NS = 16
NW = NC * NS
B = 128
NB = 80
NBUF = 4
LA = 3
E_PAD = NW * B * NB
ROWS_PER_TILE = N_PAD // NS

_AGG_COLS = []
for _j in range(DH // 32):
    _AGG_COLS += [32 * _j + 2 * _q for _q in range(16)]
    _AGG_COLS += [32 * _j + 2 * _q + 1 for _q in range(16)]


def _edge_body(xlo_hbm, xhi_hbm, s_hbm, t_hbm, src_hbm, dst_hbm, zeros_hbm,
               out_hbm,
               s_tab, t_tab, src_all, dst_all,
               g0, g1, g2, g3, s0, s1, s2, s3, agg,
               gg0, gg1, gg2, gg3, ss0, ss1, ss2, ss3):
    gbufs = (g0, g1, g2, g3)
    sbufs = (s0, s1, s2, s3)
    gsems = (gg0, gg1, gg2, gg3)
    ssems = (ss0, ss1, ss2, ss3)
    core = lax.axis_index("c")
    sid = lax.axis_index("s")
    wid = sid * NC + core

    pltpu.sync_copy(s_hbm.at[pl.ds(0, N_TAB)], s_tab)
    pltpu.sync_copy(t_hbm.at[pl.ds(0, N_TAB)], t_tab)
    pltpu.sync_copy(src_hbm.at[pl.ds(wid * NB, NB)], src_all)
    pltpu.sync_copy(dst_hbm.at[pl.ds(wid * NB, NB)], dst_all)

    himask = jnp.full((16,), -65536, jnp.int32)

    def scale_convert(gbuf, sbuf, b):
        def body(g, carry):
            si = src_all[b, pl.ds(g * 16, 16)]
            di = dst_all[b, pl.ds(g * 16, 16)]
            z = plsc.load_gather(s_tab, [si]) + plsc.load_gather(t_tab, [di])
            sv = 1.0 / (1.0 + jnp.exp(-z))
            for k in range(16):
                e = g * 16 + k
                sc = sv[k]
                for j in range(DH // 32):
                    v = plsc.bitcast(gbuf[e, pl.ds(j * 32, 32)], jnp.int32)
                    fa = plsc.bitcast(v << 16, jnp.float32)
                    fb = plsc.bitcast(v & himask, jnp.float32)
                    sbuf[e, pl.ds(j * 32, 16)] = fa * sc
                    sbuf[e, pl.ds(j * 32 + 16, 16)] = fb * sc
            return carry
        lax.fori_loop(0, B // 16, body, 0)

    rtile = pl.ds(sid * ROWS_PER_TILE, ROWS_PER_TILE)

    for half, x_hbm in enumerate((xlo_hbm, xhi_hbm)):
        def issue_gather(b, k):
            pltpu.async_copy(x_hbm.at[src_all.at[b]], gbufs[k], gsems[k])

        def wait_gather(k):
            pltpu.make_async_copy(x_hbm.at[src_all.at[0]], gbufs[k],
                                  gsems[k]).wait()

        def issue_scatter(b, k):
            pltpu.async_copy(sbufs[k], agg.at[dst_all.at[b]], ssems[k],
                             add=True)

        def wait_scatter(k):
            pltpu.make_async_copy(sbufs[k], agg.at[dst_all.at[0]],
                                  ssems[k]).wait()

        pltpu.sync_copy(zeros_hbm, agg.at[rtile])
        plsc.subcore_barrier()

        for k in range(LA):
            issue_gather(k, k)

        for k in range(NBUF):
            wait_gather(k)
            issue_gather(k + LA, (k + LA) % NBUF)
            scale_convert(gbufs[k], sbufs[k], k)
            issue_scatter(k, k)

        def round_body(t, carry):
            for k in range(NBUF):
                m = NBUF * t + k
                wait_gather(k)
                issue_gather(m + LA, (k + LA) % NBUF)
                wait_scatter(k)
                scale_convert(gbufs[k], sbufs[k], m)
                issue_scatter(m, k)
            return carry

        lax.fori_loop(1, NB // NBUF - 1, round_body, 0)

        for k in range(NBUF):
            m = NB - NBUF + k
            wait_gather(k)
            if m + LA < NB:
                issue_gather(m + LA, (k + LA) % NBUF)
            wait_scatter(k)
            scale_convert(gbufs[k], sbufs[k], m)
            issue_scatter(m, k)

        for k in range(NBUF):
            wait_scatter(k)
        plsc.subcore_barrier()

        pltpu.sync_copy(agg.at[rtile], out_hbm.at[core, half, rtile])
        plsc.subcore_barrier()


@functools.cache
def _edge_kernel_fn():
    return functools.partial(
        pl.kernel,
        out_type=jax.ShapeDtypeStruct((NC, 2, N_PAD, DH), jnp.float32),
        mesh=plsc.VectorSubcoreMesh(core_axis_name="c", subcore_axis_name="s"),
        compiler_params=pltpu.CompilerParams(needs_layout_passes=False,
                                             use_tc_tiling_on_sc=False),
        scratch_types=[
            pltpu.VMEM((N_TAB,), jnp.float32),
            pltpu.VMEM((N_TAB,), jnp.float32),
            pltpu.VMEM((NB, B), jnp.int32),
            pltpu.VMEM((NB, B), jnp.int32),
        ] + [pltpu.VMEM((B, DH), jnp.bfloat16) for _ in range(NBUF)]
          + [pltpu.VMEM((B, DH), jnp.float32) for _ in range(NBUF)]
          + [pltpu.VMEM_SHARED((N_PAD, DH), jnp.float32)]
          + [pltpu.SemaphoreType.DMA for _ in range(2 * NBUF)],
    )(_edge_body)


BLK = 1024
GRID = N_PAD // BLK


def _update_body(x_ref, p00_ref, p01_ref, p10_ref, p11_ref, w_ref, b_ref,
                 ws_ref, wt_ref, bias_ref,
                 xo_ref, xlo_ref, xhi_ref, so_ref, to_ref):
    x = x_ref[...]
    agg = jnp.concatenate(
        [p00_ref[0, 0] + p10_ref[0, 0], p01_ref[0, 0] + p11_ref[0, 0]],
        axis=1)
    h = jnp.dot(x, w_ref[0], preferred_element_type=jnp.float32)
    h = h + jnp.dot(agg, w_ref[1], preferred_element_type=jnp.float32)
    h = h + b_ref[...]
    xn = jnp.maximum(h, 0.0)
    xo_ref[...] = xn
    xlo_ref[...] = xn[:, :DH].astype(jnp.bfloat16)
    xhi_ref[...] = xn[:, DH:].astype(jnp.bfloat16)
    so_ref[...] = jnp.sum(xn * ws_ref[...], axis=1)
    to_ref[...] = jnp.sum(xn * wt_ref[...], axis=1) + bias_ref[0, 0]


def _update(x, parts, w2, bcast, ws, wt, bias):
    return pl.pallas_call(
        _update_body,
        grid=(GRID,),
        in_specs=[
            pl.BlockSpec((BLK, D), lambda i: (i, 0)),
            pl.BlockSpec((1, 1, BLK, DH), lambda i: (0, 0, i, 0)),
            pl.BlockSpec((1, 1, BLK, DH), lambda i: (0, 1, i, 0)),
            pl.BlockSpec((1, 1, BLK, DH), lambda i: (1, 0, i, 0)),
            pl.BlockSpec((1, 1, BLK, DH), lambda i: (1, 1, i, 0)),
            pl.BlockSpec((2, D, D), lambda i: (0, 0, 0)),
            pl.BlockSpec((1, D), lambda i: (0, 0)),
            pl.BlockSpec((1, D), lambda i: (0, 0)),
            pl.BlockSpec((1, D), lambda i: (0, 0)),
            pl.BlockSpec((1, 1), lambda i: (0, 0)),
        ],
        out_specs=[
            pl.BlockSpec((BLK, D), lambda i: (i, 0)),
            pl.BlockSpec((BLK, DH), lambda i: (i, 0)),
            pl.BlockSpec((BLK, DH), lambda i: (i, 0)),
            pl.BlockSpec((BLK,), lambda i: (i,)),
            pl.BlockSpec((BLK,), lambda i: (i,)),
        ],
        out_shape=[
            jax.ShapeDtypeStruct((N_PAD, D), jnp.float32),
            jax.ShapeDtypeStruct((N_PAD, DH), jnp.bfloat16),
            jax.ShapeDtypeStruct((N_PAD, DH), jnp.bfloat16),
            jax.ShapeDtypeStruct((N_PAD,), jnp.float32),
            jax.ShapeDtypeStruct((N_PAD,), jnp.float32),
        ],
    )(x, parts, parts, parts, parts, w2, bcast, ws, wt, bias)


def _proj_body(x_ref, ws_ref, wt_ref, bias_ref, so_ref, to_ref):
    x = x_ref[...]
    so_ref[...] = jnp.sum(x * ws_ref[...], axis=1)
    to_ref[...] = jnp.sum(x * wt_ref[...], axis=1) + bias_ref[0, 0]


def _proj(x, ws, wt, bias):
    return pl.pallas_call(
        _proj_body,
        grid=(GRID,),
        in_specs=[
            pl.BlockSpec((BLK, D), lambda i: (i, 0)),
            pl.BlockSpec((1, D), lambda i: (0, 0)),
            pl.BlockSpec((1, D), lambda i: (0, 0)),
            pl.BlockSpec((1, 1), lambda i: (0, 0)),
        ],
        out_specs=[
            pl.BlockSpec((BLK,), lambda i: (i,)),
            pl.BlockSpec((BLK,), lambda i: (i,)),
        ],
        out_shape=[
            jax.ShapeDtypeStruct((N_PAD,), jnp.float32),
            jax.ShapeDtypeStruct((N_PAD,), jnp.float32),
        ],
    )(x, ws, wt, bias)


def kernel(edge_index, user_emb, item_emb, att_w, att_b, agg_w, agg_b):
    src = edge_index[0]
    dst = edge_index[1]
    pad = E_PAD - E
    src_p = jnp.concatenate([src, jnp.zeros((pad,), jnp.int32)]).reshape(NW * NB, B)
    dst_p = jnp.concatenate([dst, jnp.full((pad,), DUMMY, jnp.int32)]).reshape(NW * NB, B)
    zeros = jnp.zeros((ROWS_PER_TILE, DH), jnp.float32)

    x = jnp.concatenate(
        [user_emb, item_emb, jnp.zeros((N_PAD - N, D), jnp.float32)], axis=0)
    xlo = x[:, :DH].astype(jnp.bfloat16)
    xhi = x[:, DH:].astype(jnp.bfloat16)

    ws = [att_w[l, :D, 0].reshape(1, D) for l in range(L_LAYERS)]
    wt = [att_w[l, D:, 0].reshape(1, D) for l in range(L_LAYERS)]
    bs = [att_b[l].reshape(1, 1) for l in range(L_LAYERS)]
    bc = [agg_b[l].reshape(1, D) for l in range(L_LAYERS)]

    perm = jnp.asarray(_AGG_COLS + [c + DH for c in _AGG_COLS], jnp.int32)
    w2 = [jnp.stack([agg_w[l][:D], agg_w[l][D:][perm]]) for l in range(L_LAYERS)]

    s, t = _proj(x, ws[0], wt[0], bs[0])
    for l in range(L_LAYERS):
        parts = _edge_kernel_fn()(xlo, xhi, s, t, src_p, dst_p, zeros)
        nl = min(l + 1, L_LAYERS - 1)
        x, xlo, xhi, s, t = _update(x, parts, w2[l], bc[l], ws[nl], wt[nl], bs[nl])

    return (x[:NUM_USERS], x[NUM_USERS:N])
